# Initial kernel scaffold; baseline (speedup 1.0000x reference)
import jax
import jax.numpy as jnp
from jax.experimental import pallas as pl


def kernel(x):
    raise NotImplementedError("write your pallas kernel here")


# speedup vs baseline: 7.2987x; 7.2987x over previous
import functools

import jax
import jax.numpy as jnp
from jax.experimental import pallas as pl
from jax.experimental.pallas import tpu as pltpu

_BH = 512


def _nms_body(bh, x_ref, top_ref, bot_ref, o_ref):
    i = pl.program_id(1)
    ni = pl.num_programs(1)
    cur = x_ref[0]
    top = jnp.where(i == 0, cur[0:1, :], top_ref[0, 7:8, :])
    bot = jnp.where(i == ni - 1, cur[bh - 1 : bh, :], bot_ref[0, 0:1, :])
    ext = jnp.concatenate([top, cur, bot], axis=0)
    left = jnp.concatenate([ext[:, :1], ext[:, :-1]], axis=1)
    right = jnp.concatenate([ext[:, 1:], ext[:, -1:]], axis=1)
    h2 = jnp.maximum(left, right)
    h3 = jnp.maximum(h2, ext)
    nmax = jnp.maximum(
        jnp.maximum(h3[0:bh], h3[2 : bh + 2]),
        h2[1 : bh + 1],
    )
    o_ref[0] = jnp.where(cur > nmax, cur, 0.0)


def _nms(x, *, interpret=False):
    b, c, h, w = x.shape
    bc = b * c
    xr = x.reshape(bc, h, w)
    bh = min(_BH, h)
    ni = h // bh
    g8 = h // 8
    bh8 = bh // 8

    grid = (bc, ni)
    out = pl.pallas_call(
        functools.partial(_nms_body, bh),
        out_shape=jax.ShapeDtypeStruct((bc, h, w), x.dtype),
        grid=grid,
        in_specs=[
            pl.BlockSpec((1, bh, w), lambda b_, i: (b_, i, 0)),
            pl.BlockSpec(
                (1, 8, w), lambda b_, i: (b_, jnp.maximum(i * bh8 - 1, 0), 0)
            ),
            pl.BlockSpec(
                (1, 8, w),
                lambda b_, i: (b_, jnp.minimum((i + 1) * bh8, g8 - 1), 0),
            ),
        ],
        out_specs=pl.BlockSpec((1, bh, w), lambda b_, i: (b_, i, 0)),
        compiler_params=pltpu.CompilerParams(
            dimension_semantics=("parallel", "arbitrary"),
            vmem_limit_bytes=48 * 1024 * 1024,
        ),
        name="nms2d",
        interpret=interpret,
    )(xr, xr, xr)
    return out.reshape(b, c, h, w)


def kernel(x):
    return _nms(x)
